# Initial kernel scaffold; baseline (speedup 1.0000x reference)
import jax
import jax.numpy as jnp
from jax.experimental import pallas as pl


def kernel(x, edge_index, W_l1, b_l1, W_r1, bn1_g, bn1_b, W_l2, b_l2, W_r2, bn2_g, bn2_b, W_fc, b_fc):
    raise NotImplementedError("write your pallas kernel here")


# speedup vs baseline: 4.1880x; 4.1880x over previous
import functools

import jax
import jax.numpy as jnp
from jax import lax
from jax.experimental import pallas as pl
from jax.experimental.pallas import tpu as pltpu
from jax.experimental.pallas import tpu_sc as plsc

N = 10000
D = 128
NC = 2
NS = 16
NW = NC * NS
K = 128
RPT = 632
RPAD = NS * RPT
TRASH = N


def _sc_agg(E_pad, with_deg):
    chunks = E_pad // (NW * K)
    ept = chunks * K
    mesh = plsc.VectorSubcoreMesh(core_axis_name="c", subcore_axis_name="s")

    out_type = [jax.ShapeDtypeStruct((NC, RPAD, D), jnp.float32)]
    scratch = [
        pltpu.VMEM((K,), jnp.int32),
        pltpu.VMEM((K,), jnp.int32),
        pltpu.VMEM((K, D), jnp.float32),
        pltpu.VMEM_SHARED((RPAD, D), jnp.float32),
        pltpu.SemaphoreType.DMA,
    ]
    if with_deg:
        out_type.append(jax.ShapeDtypeStruct((NC * RPAD,), jnp.float32))
        scratch.append(pltpu.VMEM((K,), jnp.float32))
        scratch.append(pltpu.VMEM_SHARED((RPAD,), jnp.float32))
        scratch.append(pltpu.VMEM((RPT,), jnp.float32))

    def body(x_hbm, src_hbm, dst_hbm, z2_hbm, z1_hbm, *rest):
        if with_deg:
            (acc_out, deg_out, src_v, dst_v, rows_v, acc_s, gsem,
             ones_v, deg_s, deg_stage) = rest
        else:
            acc_out, src_v, dst_v, rows_v, acc_s, gsem = rest
        c = lax.axis_index("c")
        s = lax.axis_index("s")
        wid = s * NC + c

        pltpu.sync_copy(z2_hbm.at[pl.ds(s * RPT, RPT)],
                        acc_s.at[pl.ds(s * RPT, RPT)])
        if with_deg:
            pltpu.sync_copy(z1_hbm.at[pl.ds(s * RPT, RPT)], deg_stage)
            pltpu.sync_copy(deg_stage, deg_s.at[pl.ds(s * RPT, RPT)])
            for j in range(K // 16):
                ones_v[pl.ds(j * 16, 16)] = jnp.ones((16,), jnp.float32)
        plsc.subcore_barrier()

        base0 = wid * ept

        def step(i, carry):
            base = base0 + i * K
            pltpu.sync_copy(src_hbm.at[pl.ds(base, K)], src_v)
            pltpu.sync_copy(dst_hbm.at[pl.ds(base, K)], dst_v)
            pltpu.async_copy(x_hbm.at[src_v], rows_v, gsem).wait()
            pltpu.sync_copy(rows_v, acc_s.at[dst_v], add=True)
            if with_deg:
                pltpu.sync_copy(ones_v, deg_s.at[dst_v], add=True)
            return carry

        lax.fori_loop(0, chunks, step, 0)
        plsc.subcore_barrier()

        pltpu.sync_copy(acc_s.at[pl.ds(s * RPT, RPT)],
                        acc_out.at[c, pl.ds(s * RPT, RPT)])
        if with_deg:
            pltpu.sync_copy(deg_s.at[pl.ds(s * RPT, RPT)], deg_stage)
            pltpu.sync_copy(deg_stage,
                            deg_out.at[pl.ds(c * RPAD + s * RPT, RPT)])

    return pl.kernel(body, out_type=out_type, mesh=mesh,
                     scratch_types=scratch)


def _dot_t(a, b):
    return lax.dot_general(a, b, (((1,), (1,)), ((), ())),
                           preferred_element_type=jnp.float32)


def _tc1_body(acc_ref, invd_ref, x_ref, wl_ref, bl_ref, wr_ref,
              g_ref, b_ref, out_ref):
    aggsum = acc_ref[0, :N, :] + acc_ref[1, :N, :]
    agg = aggsum * invd_ref[...]
    p = _dot_t(agg, wl_ref[...]) + bl_ref[...] + _dot_t(x_ref[...], wr_ref[...])
    mu = jnp.mean(p, axis=0, keepdims=True)
    var = jnp.mean((p - mu) ** 2, axis=0, keepdims=True)
    h = (p - mu) * lax.rsqrt(var + 1e-5) * g_ref[...] + b_ref[...]
    out_ref[...] = jnp.maximum(h, 0.0)


def _tc2_body(acc_ref, invd_ref, h_ref, wl_ref, bl_ref, wr_ref,
              g_ref, b_ref, wfc_ref, bfc_ref, out_ref):
    aggsum = acc_ref[0, :N, :] + acc_ref[1, :N, :]
    agg = aggsum * invd_ref[...]
    p = _dot_t(agg, wl_ref[...]) + bl_ref[...] + _dot_t(h_ref[...], wr_ref[...])
    mu = jnp.mean(p, axis=0, keepdims=True)
    var = jnp.mean((p - mu) ** 2, axis=0, keepdims=True)
    h2 = (p - mu) * lax.rsqrt(var + 1e-5) * g_ref[...] + b_ref[...]
    h2 = jnp.maximum(h2, 0.0)
    out_ref[...] = _dot_t(h2, wfc_ref[...]) + bfc_ref[...]


def kernel(x, edge_index, W_l1, b_l1, W_r1, bn1_g, bn1_b,
           W_l2, b_l2, W_r2, bn2_g, bn2_b, W_fc, b_fc):
    E = edge_index.shape[1]
    pad = (-E) % (NW * K)
    E_pad = E + pad
    src_p = jnp.concatenate(
        [edge_index[0], jnp.zeros((pad,), jnp.int32)])
    dst_p = jnp.concatenate(
        [edge_index[1], jnp.full((pad,), TRASH, jnp.int32)])
    z2 = jnp.zeros((RPAD, D), jnp.float32)
    z1 = jnp.zeros((RPAD,), jnp.float32)

    acc1, degp = _sc_agg(E_pad, True)(x, src_p, dst_p, z2, z1)
    deg = degp[:N] + degp[RPAD:RPAD + N]
    inv_deg = (1.0 / jnp.maximum(deg, 1.0)).reshape(N, 1)

    h1 = pl.pallas_call(
        _tc1_body,
        out_shape=jax.ShapeDtypeStruct((N, D), jnp.float32),
    )(acc1, inv_deg, x, W_l1, b_l1.reshape(1, D), W_r1,
      bn1_g.reshape(1, D), bn1_b.reshape(1, D))

    (acc2,) = _sc_agg(E_pad, False)(h1, src_p, dst_p, z2, z1)

    C = W_fc.shape[0]
    out = pl.pallas_call(
        _tc2_body,
        out_shape=jax.ShapeDtypeStruct((N, C), jnp.float32),
    )(acc2, inv_deg, h1, W_l2, b_l2.reshape(1, D), W_r2,
      bn2_g.reshape(1, D), bn2_b.reshape(1, D), W_fc, b_fc.reshape(1, C))
    return out
